# baseline (speedup 1.0000x reference)
import functools
import math

import jax
import jax.numpy as jnp
from jax import lax
from jax.experimental import pallas as pl
from jax.experimental.pallas import tpu as pltpu
from jax.experimental.pallas import tpu_sc as plsc

_VOCAB = 100000
_D = 512
_B = 4
_S = 2048
_SCALAR = 10000.0

_NC = 2
_NS = 16
_NW = _NC * _NS
_CHUNK = _S // _NW
_LANES = 16
_QR = 64
_QB = _S // _QR


def _pos_body(o_ref):
    c8 = lax.broadcasted_iota(jnp.int32, (8, _D), 1)
    within = c8 & 31
    j8 = (c8 - within) + (c8 & 1) * 16 + (within >> 1)
    inv_freq = jnp.exp(j8.astype(jnp.float32) * (-2.0 * math.log(_SCALAR) / _D))[0:1, :]
    phase = jnp.where((j8 & 1) == 0, 0.0, 0.5 * math.pi).astype(jnp.float32)[0:1, :]
    r = lax.broadcasted_iota(jnp.int32, (_QR, _D), 0).astype(jnp.float32)
    arg_r = r * inv_freq + phase
    sin_r = jnp.sin(arg_r)
    cos_r = jnp.cos(arg_r)
    q = lax.broadcasted_iota(jnp.int32, (_QB, _D), 0).astype(jnp.float32)
    arg_q = (q * float(_QR)) * inv_freq
    sin_q = jnp.sin(arg_q)
    cos_q = jnp.cos(arg_q)
    for qi in range(_QB):
        o_ref[pl.ds(qi * _QR, _QR), :] = (
            sin_r * cos_q[qi:qi + 1, :] + cos_r * sin_q[qi:qi + 1, :]
        ).astype(jnp.bfloat16)


_pos_table = pl.pallas_call(
    _pos_body,
    out_shape=jax.ShapeDtypeStruct((_S, _D), jnp.bfloat16),
)


def _sc_embed_body(x_hbm, table_hbm, pos_hbm, out_hbm,
                   idx_v, pos_v, r0, r1,
                   g0, g1, s0, s1, isem, psem):
    wid = lax.axis_index("s") * _NC + lax.axis_index("c")
    s_base = wid * _CHUNK

    idx_h = [None] * _B
    for b in range(_B):
        idx_h[b] = pltpu.async_copy(
            x_hbm.at[b, pl.ds(s_base, _CHUNK)], idx_v.at[b], isem)
    pos_h = pltpu.async_copy(pos_hbm.at[pl.ds(s_base, _CHUNK)], pos_v, psem)
    for b in range(_B):
        idx_h[b].wait()

    bufs = (r0, r1)
    gsems = (g0, g1)
    ssems = (s0, s1)
    gather_h = [None] * _B
    store_h = [None] * _B
    gather_h[0] = pltpu.async_copy(table_hbm.at[idx_v.at[0]], r0, g0)
    pos_h.wait()

    for b in range(_B):
        k = b % 2
        rv = bufs[k]
        gather_h[b].wait()
        if b + 1 < _B:
            if b >= 1:
                store_h[b - 1].wait()
            gather_h[b + 1] = pltpu.async_copy(
                table_hbm.at[idx_v.at[b + 1]], bufs[(b + 1) % 2],
                gsems[(b + 1) % 2])

        def _row_add(i, _, rv=rv):
            for j2 in range(_D // (2 * _LANES)):
                base = j2 * 2 * _LANES
                w = pos_v[i, pl.ds(j2 * _LANES, _LANES)]
                lo = lax.bitcast_convert_type(
                    lax.shift_left(w, 16), jnp.float32)
                hi = lax.bitcast_convert_type(
                    w & jnp.int32(-65536), jnp.float32)
                sl = pl.ds(base, _LANES)
                sh = pl.ds(base + _LANES, _LANES)
                rv[i, sl] = rv[i, sl] + lo
                rv[i, sh] = rv[i, sh] + hi
            return 0

        lax.fori_loop(0, _CHUNK, _row_add, 0)
        store_h[b] = pltpu.async_copy(
            rv, out_hbm.at[pl.ds(b * _S + s_base, _CHUNK)], ssems[k])
    store_h[_B - 2].wait()
    store_h[_B - 1].wait()


@functools.lru_cache(maxsize=None)
def _get_sc_embed():
    return functools.partial(
        pl.kernel,
        mesh=plsc.VectorSubcoreMesh(core_axis_name="c", subcore_axis_name="s"),
        out_type=jax.ShapeDtypeStruct((_B * _S, _D), jnp.float32),
        scratch_types=(
            [pltpu.VMEM((_B, _CHUNK), jnp.int32),
             pltpu.VMEM((_CHUNK, _D // 2), jnp.int32),
             pltpu.VMEM((_CHUNK, _D), jnp.float32),
             pltpu.VMEM((_CHUNK, _D), jnp.float32)]
            + [pltpu.SemaphoreType.DMA] * 6
        ),
    )(_sc_embed_body)


def kernel(x, table):
    _sc_embed = _get_sc_embed()
    pos16 = _pos_table()
    pos_i32 = lax.bitcast_convert_type(
        pos16.reshape(_S, _D // 2, 2), jnp.int32)
    out = _sc_embed(x.astype(jnp.int32), table, pos_i32)
    return out.reshape(_B, _S, _D)


# speedup vs baseline: 1.8353x; 1.8353x over previous
import functools
import math

import jax
import jax.numpy as jnp
from jax import lax
from jax.experimental import pallas as pl
from jax.experimental.pallas import tpu as pltpu
from jax.experimental.pallas import tpu_sc as plsc

_VOCAB = 100000
_D = 512
_B = 4
_S = 2048
_SCALAR = 10000.0

_NC = 2
_NS = 16
_NW = _NC * _NS
_CHUNK = _S // _NW
_LANES = 16
_QR = 64
_QB = _S // _QR


def _pos_body(o_ref):
    j8 = lax.broadcasted_iota(jnp.int32, (8, _D), 1)
    inv_freq = jnp.exp(j8.astype(jnp.float32) * (-2.0 * math.log(_SCALAR) / _D))[0:1, :]
    phase = jnp.where((j8 & 1) == 0, 0.0, 0.5 * math.pi).astype(jnp.float32)[0:1, :]
    r = lax.broadcasted_iota(jnp.int32, (_QR, _D), 0).astype(jnp.float32)
    arg_r = r * inv_freq + phase
    sin_r = jnp.sin(arg_r)
    cos_r = jnp.cos(arg_r)
    q = lax.broadcasted_iota(jnp.int32, (_QB, _D), 0).astype(jnp.float32)
    arg_q = (q * float(_QR)) * inv_freq
    sin_q = jnp.sin(arg_q)
    cos_q = jnp.cos(arg_q)
    for qi in range(_QB):
        o_ref[pl.ds(qi * _QR, _QR), :] = (
            sin_r * cos_q[qi:qi + 1, :] + cos_r * sin_q[qi:qi + 1, :])


_pos_table = pl.pallas_call(
    _pos_body,
    out_shape=jax.ShapeDtypeStruct((_S, _D), jnp.float32),
)


def _sc_embed_body(x_hbm, table_hbm, pos_hbm, out_hbm,
                   idx_v, pos_v, r0, r1,
                   g0, g1, s0, s1, isem, psem):
    wid = lax.axis_index("s") * _NC + lax.axis_index("c")
    s_base = wid * _CHUNK

    idx_h = [None] * _B
    for b in range(_B):
        idx_h[b] = pltpu.async_copy(
            x_hbm.at[b, pl.ds(s_base, _CHUNK)], idx_v.at[b], isem)
    pos_h = pltpu.async_copy(pos_hbm.at[pl.ds(s_base, _CHUNK)], pos_v, psem)
    for b in range(_B):
        idx_h[b].wait()

    bufs = (r0, r1)
    gsems = (g0, g1)
    ssems = (s0, s1)
    gather_h = [None] * _B
    store_h = [None] * _B
    gather_h[0] = pltpu.async_copy(table_hbm.at[idx_v.at[0]], r0, g0)
    pos_h.wait()

    for b in range(_B):
        k = b % 2
        rv = bufs[k]
        gather_h[b].wait()
        if b + 1 < _B:
            if b >= 1:
                store_h[b - 1].wait()
            gather_h[b + 1] = pltpu.async_copy(
                table_hbm.at[idx_v.at[b + 1]], bufs[(b + 1) % 2],
                gsems[(b + 1) % 2])

        def _row_add(i, _, rv=rv):
            for j in range(_D // _LANES):
                sl = pl.ds(j * _LANES, _LANES)
                rv[i, sl] = rv[i, sl] + pos_v[i, sl]
            return 0

        lax.fori_loop(0, _CHUNK, _row_add, 0)
        store_h[b] = pltpu.async_copy(
            rv, out_hbm.at[pl.ds(b * _S + s_base, _CHUNK)], ssems[k])
    store_h[_B - 2].wait()
    store_h[_B - 1].wait()


@functools.lru_cache(maxsize=None)
def _get_sc_embed():
    return functools.partial(
        pl.kernel,
        mesh=plsc.VectorSubcoreMesh(core_axis_name="c", subcore_axis_name="s"),
        out_type=jax.ShapeDtypeStruct((_B * _S, _D), jnp.float32),
        scratch_types=(
            [pltpu.VMEM((_B, _CHUNK), jnp.int32),
             pltpu.VMEM((_CHUNK, _D), jnp.float32),
             pltpu.VMEM((_CHUNK, _D), jnp.float32),
             pltpu.VMEM((_CHUNK, _D), jnp.float32)]
            + [pltpu.SemaphoreType.DMA] * 6
        ),
    )(_sc_embed_body)


def kernel(x, table):
    _sc_embed = _get_sc_embed()
    pos = _pos_table()
    out = _sc_embed(x.astype(jnp.int32), table, pos)
    return out.reshape(_B, _S, _D)


# speedup vs baseline: 1.8445x; 1.0050x over previous
import functools
import math

import jax
import jax.numpy as jnp
from jax import lax
from jax.experimental import pallas as pl
from jax.experimental.pallas import tpu as pltpu
from jax.experimental.pallas import tpu_sc as plsc

_VOCAB = 100000
_D = 512
_B = 4
_S = 2048
_SCALAR = 10000.0

_NC = 2
_NS = 16
_NW = _NC * _NS
_CHUNK = _S // _NW
_LANES = 16
_QR = 64
_QB = _S // _QR


def _pos_body(o_ref):
    j8 = lax.broadcasted_iota(jnp.int32, (8, _D), 1)
    inv_freq = jnp.exp(j8.astype(jnp.float32) * (-2.0 * math.log(_SCALAR) / _D))[0:1, :]
    phase = jnp.where((j8 & 1) == 0, 0.0, 0.5 * math.pi).astype(jnp.float32)[0:1, :]
    r = lax.broadcasted_iota(jnp.int32, (_QR, _D), 0).astype(jnp.float32)
    arg_r = r * inv_freq + phase
    sin_r = jnp.sin(arg_r)
    cos_r = jnp.cos(arg_r)
    q = lax.broadcasted_iota(jnp.int32, (_QB, _D), 0).astype(jnp.float32)
    arg_q = (q * float(_QR)) * inv_freq
    sin_q = jnp.sin(arg_q)
    cos_q = jnp.cos(arg_q)
    for qi in range(_QB):
        o_ref[pl.ds(qi * _QR, _QR), :] = (
            sin_r * cos_q[qi:qi + 1, :] + cos_r * sin_q[qi:qi + 1, :])


_pos_table = pl.pallas_call(
    _pos_body,
    out_shape=jax.ShapeDtypeStruct((_S, _D), jnp.float32),
)


def _sc_embed_body(x_hbm, table_hbm, pos_hbm, out_hbm,
                   idx_v, pos_v, r0, r1,
                   g0, g1, s0, s1, isem, psem):
    wid = lax.axis_index("s") * _NC + lax.axis_index("c")
    s_base = wid * _CHUNK

    idx_h = [None] * _B
    for b in range(_B):
        idx_h[b] = pltpu.async_copy(
            x_hbm.at[b, pl.ds(s_base, _CHUNK)], idx_v.at[b], isem)
    pos_h = pltpu.async_copy(pos_hbm.at[pl.ds(s_base, _CHUNK)], pos_v, psem)
    for b in range(_B):
        idx_h[b].wait()

    bufs = (r0, r1)
    gsems = (g0, g1)
    ssems = (s0, s1)
    gather_h = [None] * _B
    store_h = [None] * _B
    gather_h[0] = pltpu.async_copy(table_hbm.at[idx_v.at[0]], r0, g0)
    pos_h.wait()

    half = _CHUNK // 2
    for b in range(_B):
        k = b % 2
        rv = bufs[k]
        gather_h[b].wait()
        if b + 1 < _B:
            if b >= 1:
                for h in store_h[b - 1]:
                    h.wait()
            gather_h[b + 1] = pltpu.async_copy(
                table_hbm.at[idx_v.at[b + 1]], bufs[(b + 1) % 2],
                gsems[(b + 1) % 2])

        def _row_add(i, _, rv=rv):
            for j in range(_D // _LANES):
                sl = pl.ds(j * _LANES, _LANES)
                rv[i, sl] = rv[i, sl] + pos_v[i, sl]
            return 0

        lax.fori_loop(0, half, _row_add, 0)
        st0 = pltpu.async_copy(
            rv.at[pl.ds(0, half)],
            out_hbm.at[pl.ds(b * _S + s_base, half)], ssems[k])
        lax.fori_loop(half, _CHUNK, _row_add, 0)
        st1 = pltpu.async_copy(
            rv.at[pl.ds(half, half)],
            out_hbm.at[pl.ds(b * _S + s_base + half, half)], ssems[k])
        store_h[b] = (st0, st1)
    for h in store_h[_B - 2] + store_h[_B - 1]:
        h.wait()


@functools.lru_cache(maxsize=None)
def _get_sc_embed():
    return functools.partial(
        pl.kernel,
        mesh=plsc.VectorSubcoreMesh(core_axis_name="c", subcore_axis_name="s"),
        out_type=jax.ShapeDtypeStruct((_B * _S, _D), jnp.float32),
        scratch_types=(
            [pltpu.VMEM((_B, _CHUNK), jnp.int32),
             pltpu.VMEM((_CHUNK, _D), jnp.float32),
             pltpu.VMEM((_CHUNK, _D), jnp.float32),
             pltpu.VMEM((_CHUNK, _D), jnp.float32)]
            + [pltpu.SemaphoreType.DMA] * 6
        ),
    )(_sc_embed_body)


def kernel(x, table):
    _sc_embed = _get_sc_embed()
    pos = _pos_table()
    out = _sc_embed(x.astype(jnp.int32), table, pos)
    return out.reshape(_B, _S, _D)
